# baseline (speedup 1.0000x reference)
import functools

import jax
import jax.numpy as jnp
from jax import lax
from jax.experimental import pallas as pl
from jax.experimental.pallas import tpu as pltpu
from jax.experimental.pallas import tpu_sc as plsc

N = 10000
K = 32
D = 128
LANES = 8

NS = 16
NW = 2 * NS
BN = 40
NBLK = 8
N_PAD = NW * NBLK * BN

_INV_K = 1.0 / K


def _mm_at(a, b):
    return lax.dot_general(a, b, (((1,), (1,)), ((), ())),
                           preferred_element_type=jnp.float32)


def _stage_a_body(x_ref, w1_ref, wg1_ref, o_ref):
    h = jnp.maximum(_mm_at(x_ref[...], w1_ref[...]), 0.0)
    o_ref[...] = _mm_at(h, wg1_ref[...])


def _stage_a(x, w1, wg1):
    blk = 1000
    grid = N // blk
    return pl.pallas_call(
        _stage_a_body,
        grid=(grid,),
        in_specs=[
            pl.BlockSpec((blk, D), lambda i: (i, 0)),
            pl.BlockSpec((D, D), lambda i: (0, 0)),
            pl.BlockSpec((D, D), lambda i: (0, 0)),
        ],
        out_specs=pl.BlockSpec((blk, D), lambda i: (i, 0)),
        out_shape=jax.ShapeDtypeStruct((N, D), jnp.float32),
    )(x, w1, wg1)


def _stage_b_body(s_ref, bg_ref, wg2_ref, o_ref):
    a = s_ref[...] * _INV_K + bg_ref[...]
    h = jnp.maximum(a, 0.0)
    o_ref[...] = _mm_at(h, wg2_ref[...])


def _stage_b(s1, bg1, wg2):
    blk = 1000
    grid = N // blk
    return pl.pallas_call(
        _stage_b_body,
        grid=(grid,),
        in_specs=[
            pl.BlockSpec((blk, D), lambda i: (i, 0)),
            pl.BlockSpec((1, D), lambda i: (0, 0)),
            pl.BlockSpec((D, D), lambda i: (0, 0)),
        ],
        out_specs=pl.BlockSpec((blk, D), lambda i: (i, 0)),
        out_shape=jax.ShapeDtypeStruct((N, D), jnp.float32),
    )(s1, bg1.reshape(1, D), wg2)


def _stage_c_body(s_ref, bg_ref, o_ref):
    a = s_ref[...] * _INV_K + bg_ref[...]
    nrm = jnp.sqrt(jnp.sum(a * a, axis=1, keepdims=True))
    o_ref[...] = a / nrm


def _stage_c(s2, bg2):
    blk = 1000
    grid = N // blk
    return pl.pallas_call(
        _stage_c_body,
        grid=(grid,),
        in_specs=[
            pl.BlockSpec((blk, D), lambda i: (i, 0)),
            pl.BlockSpec((1, D), lambda i: (0, 0)),
        ],
        out_specs=pl.BlockSpec((blk, D), lambda i: (i, 0)),
        out_shape=jax.ShapeDtypeStruct((N, D), jnp.float32),
    )(s2, bg2.reshape(1, D))


def _sc_body(table_hbm, connt_hbm, out_hbm,
             idx_v, acc0, acc1, acc2, tab_s,
             gsem0, gsem1, gsem2, osem):
    cid = lax.axis_index("c")
    sid = lax.axis_index("s")
    wid = sid * 2 + cid
    accs = tuple((acc0, acc1, acc2)[b % 3] for b in range(NBLK))
    gsems = tuple((gsem0, gsem1, gsem2)[b % 3] for b in range(NBLK))

    pltpu.sync_copy(connt_hbm.at[:, wid], idx_v)

    @pl.when(sid < 10)
    def _():
        pltpu.sync_copy(table_hbm.at[pl.ds(sid * 1000, 1000)],
                        tab_s.at[pl.ds(sid * 1000, 1000)])

    plsc.subcore_barrier()

    def zero(accb):
        @pl.loop(0, BN)
        def _(r):
            for c in range(LANES):
                accb[r, pl.ds(c * 16, 16)] = jnp.zeros((16,), jnp.float32)

    def issue(b):
        @pl.loop(0, K)
        def _(k):
            pltpu.async_copy(
                tab_s.at[idx_v.at[k, b]],
                accs[b], gsems[b], add=True)

    def drain(b):
        @pl.loop(0, K)
        def _(k):
            pltpu.make_async_copy(
                tab_s.at[idx_v.at[0, 0]],
                accs[b], gsems[b]).wait()

    first = wid * NBLK
    NREAL = N // BN

    for b in range(3):
        @pl.when(first + b < NREAL)
        def _():
            zero(accs[b])
            issue(b)

    for b in range(NBLK):
        @pl.when(first + b < NREAL)
        def _():
            drain(b)
            pltpu.async_copy(
                accs[b],
                out_hbm.at[pl.ds((first + b) * BN, BN)],
                osem).wait()

        if b + 3 < NBLK:
            @pl.when(first + b + 3 < NREAL)
            def _():
                zero(accs[b + 3])
                issue(b + 3)


@functools.cache
def _sc_gather_sum():
    return pl.kernel(
        _sc_body,
        out_type=jax.ShapeDtypeStruct((N, D), jnp.float32),
        mesh=plsc.VectorSubcoreMesh(core_axis_name="c", subcore_axis_name="s",
                                    num_cores=2, num_subcores=16),
        scratch_types=[
            pltpu.VMEM((K, NBLK, BN), jnp.int32),
            pltpu.VMEM((BN, D), jnp.float32),
            pltpu.VMEM((BN, D), jnp.float32),
            pltpu.VMEM((BN, D), jnp.float32),
            pltpu.VMEM_SHARED((N, D), jnp.float32),
            pltpu.SemaphoreType.DMA,
            pltpu.SemaphoreType.DMA,
            pltpu.SemaphoreType.DMA,
            pltpu.SemaphoreType.DMA,
        ],
    )


def kernel(x, conn, W1, Wg1, bg1, Wg2, bg2):
    conn32 = conn.astype(jnp.int32)
    conn_t = jnp.pad(conn32.T, ((0, 0), (0, N_PAD - N))).reshape(
        K, NW, NBLK, BN)

    sc_gather = _sc_gather_sum()
    g1 = _stage_a(x, W1, Wg1)
    s1 = sc_gather(g1, conn_t)
    g2 = _stage_b(s1, bg1, Wg2)
    s2 = sc_gather(g2, conn_t)
    return _stage_c(s2, bg2)


# speedup vs baseline: 1.0622x; 1.0622x over previous
import functools

import jax
import jax.numpy as jnp
from jax import lax
from jax.experimental import pallas as pl
from jax.experimental.pallas import tpu as pltpu
from jax.experimental.pallas import tpu_sc as plsc

N = 10000
K = 32
D = 128
LANES = 8

NS = 16
NW = 2 * NS
BN = 80
NBLK = 4
N_PAD = NW * NBLK * BN

_INV_K = 1.0 / K


def _mm_at(a, b):
    return lax.dot_general(a, b, (((1,), (1,)), ((), ())),
                           preferred_element_type=jnp.float32)


def _stage_a_body(x_ref, w1_ref, wg1_ref, o_ref):
    h = jnp.maximum(_mm_at(x_ref[...], w1_ref[...]), 0.0)
    o_ref[...] = _mm_at(h, wg1_ref[...])


def _stage_a(x, w1, wg1):
    blk = 2000
    grid = N // blk
    return pl.pallas_call(
        _stage_a_body,
        grid=(grid,),
        in_specs=[
            pl.BlockSpec((blk, D), lambda i: (i, 0)),
            pl.BlockSpec((D, D), lambda i: (0, 0)),
            pl.BlockSpec((D, D), lambda i: (0, 0)),
        ],
        out_specs=pl.BlockSpec((blk, D), lambda i: (i, 0)),
        out_shape=jax.ShapeDtypeStruct((N, D), jnp.float32),
    )(x, w1, wg1)


def _stage_b_body(s_ref, bg_ref, wg2_ref, o_ref):
    a = s_ref[...] * _INV_K + bg_ref[...]
    h = jnp.maximum(a, 0.0)
    o_ref[...] = _mm_at(h, wg2_ref[...])


def _stage_b(s1, bg1, wg2):
    blk = 2000
    grid = N // blk
    return pl.pallas_call(
        _stage_b_body,
        grid=(grid,),
        in_specs=[
            pl.BlockSpec((blk, D), lambda i: (i, 0)),
            pl.BlockSpec((1, D), lambda i: (0, 0)),
            pl.BlockSpec((D, D), lambda i: (0, 0)),
        ],
        out_specs=pl.BlockSpec((blk, D), lambda i: (i, 0)),
        out_shape=jax.ShapeDtypeStruct((N, D), jnp.float32),
    )(s1, bg1.reshape(1, D), wg2)


def _stage_c_body(s_ref, bg_ref, o_ref):
    a = s_ref[...] * _INV_K + bg_ref[...]
    nrm = jnp.sqrt(jnp.sum(a * a, axis=1, keepdims=True))
    o_ref[...] = a / nrm


def _stage_c(s2, bg2):
    blk = 2000
    grid = N // blk
    return pl.pallas_call(
        _stage_c_body,
        grid=(grid,),
        in_specs=[
            pl.BlockSpec((blk, D), lambda i: (i, 0)),
            pl.BlockSpec((1, D), lambda i: (0, 0)),
        ],
        out_specs=pl.BlockSpec((blk, D), lambda i: (i, 0)),
        out_shape=jax.ShapeDtypeStruct((N, D), jnp.float32),
    )(s2, bg2.reshape(1, D))


def _sc_body(table_hbm, connt_hbm, out_hbm,
             idx_v, acc0, acc1, acc2, tab_s,
             gsem0, gsem1, gsem2, osem):
    cid = lax.axis_index("c")
    sid = lax.axis_index("s")
    wid = sid * 2 + cid
    accs = (acc0, acc1, acc2, acc0)
    gsems = (gsem0, gsem1, gsem2, gsem0)

    pltpu.sync_copy(connt_hbm.at[:, wid], idx_v)

    @pl.when(sid < 10)
    def _():
        pltpu.sync_copy(table_hbm.at[pl.ds(sid * 1000, 1000)],
                        tab_s.at[pl.ds(sid * 1000, 1000)])

    plsc.subcore_barrier()

    def zero(accb):
        @pl.loop(0, BN)
        def _(r):
            for c in range(LANES):
                accb[r, pl.ds(c * 16, 16)] = jnp.zeros((16,), jnp.float32)

    def issue(b):
        @pl.loop(0, K)
        def _(k):
            pltpu.async_copy(
                tab_s.at[idx_v.at[k, b]],
                accs[b], gsems[b], add=True)

    def drain(b):
        @pl.loop(0, K)
        def _(k):
            pltpu.make_async_copy(
                tab_s.at[idx_v.at[0, 0]],
                accs[b], gsems[b]).wait()

    first = wid * NBLK
    NREAL = N // BN

    for b in range(3):
        @pl.when(first + b < NREAL)
        def _():
            zero(accs[b])
            issue(b)

    for b in range(NBLK):
        @pl.when(first + b < NREAL)
        def _():
            drain(b)
            pltpu.async_copy(
                accs[b],
                out_hbm.at[pl.ds((first + b) * BN, BN)],
                osem).wait()

        if b == 0:
            @pl.when(first + 3 < NREAL)
            def _():
                zero(accs[3])
                issue(3)


@functools.cache
def _sc_gather_sum():
    return pl.kernel(
        _sc_body,
        out_type=jax.ShapeDtypeStruct((N, D), jnp.float32),
        mesh=plsc.VectorSubcoreMesh(core_axis_name="c", subcore_axis_name="s",
                                    num_cores=2, num_subcores=16),
        scratch_types=[
            pltpu.VMEM((K, NBLK, BN), jnp.int32),
            pltpu.VMEM((BN, D), jnp.float32),
            pltpu.VMEM((BN, D), jnp.float32),
            pltpu.VMEM((BN, D), jnp.float32),
            pltpu.VMEM_SHARED((N, D), jnp.float32),
            pltpu.SemaphoreType.DMA,
            pltpu.SemaphoreType.DMA,
            pltpu.SemaphoreType.DMA,
            pltpu.SemaphoreType.DMA,
        ],
    )


def kernel(x, conn, W1, Wg1, bg1, Wg2, bg2):
    conn32 = conn.astype(jnp.int32)

    conn_t = jnp.pad(conn32.T, ((0, 0), (0, N_PAD - N))).reshape(
        K, NW, NBLK, BN)

    sc_gather = _sc_gather_sum()
    g1 = _stage_a(x, W1, Wg1)
    s1 = sc_gather(g1, conn_t)
    g2 = _stage_b(s1, bg1, Wg2)
    s2 = sc_gather(g2, conn_t)
    return _stage_c(s2, bg2)


# speedup vs baseline: 1.0977x; 1.0334x over previous
import functools

import jax
import jax.numpy as jnp
from jax import lax
from jax.experimental import pallas as pl
from jax.experimental.pallas import tpu as pltpu
from jax.experimental.pallas import tpu_sc as plsc

N = 10000
K = 32
D = 128
LANES = 8

NS = 16
NW = 2 * NS
BN = 80
NBLK = 4
N_PAD = NW * NBLK * BN

_INV_K = 1.0 / K


def _mm_at(a, b):
    return lax.dot_general(a, b, (((1,), (1,)), ((), ())),
                           preferred_element_type=jnp.float32)


def _stage_a_body(x_ref, w1_ref, wg1_ref, o_ref):
    h = jnp.maximum(_mm_at(x_ref[...], w1_ref[...]), 0.0)
    o_ref[...] = _mm_at(h, wg1_ref[...])


def _stage_a(x, w1, wg1):
    blk = 5000
    grid = N // blk
    return pl.pallas_call(
        _stage_a_body,
        grid=(grid,),
        in_specs=[
            pl.BlockSpec((blk, D), lambda i: (i, 0)),
            pl.BlockSpec((D, D), lambda i: (0, 0)),
            pl.BlockSpec((D, D), lambda i: (0, 0)),
        ],
        out_specs=pl.BlockSpec((blk, D), lambda i: (i, 0)),
        out_shape=jax.ShapeDtypeStruct((N, D), jnp.float32),
    )(x, w1, wg1)


def _stage_b_body(s_ref, bg_ref, wg2_ref, o_ref):
    a = s_ref[...] * _INV_K + bg_ref[...]
    h = jnp.maximum(a, 0.0)
    o_ref[...] = _mm_at(h, wg2_ref[...])


def _stage_b(s1, bg1, wg2):
    blk = 5000
    grid = N // blk
    return pl.pallas_call(
        _stage_b_body,
        grid=(grid,),
        in_specs=[
            pl.BlockSpec((blk, D), lambda i: (i, 0)),
            pl.BlockSpec((1, D), lambda i: (0, 0)),
            pl.BlockSpec((D, D), lambda i: (0, 0)),
        ],
        out_specs=pl.BlockSpec((blk, D), lambda i: (i, 0)),
        out_shape=jax.ShapeDtypeStruct((N, D), jnp.float32),
    )(s1, bg1.reshape(1, D), wg2)


def _stage_c_body(s_ref, bg_ref, o_ref):
    a = s_ref[...] * _INV_K + bg_ref[...]
    nrm = jnp.sqrt(jnp.sum(a * a, axis=1, keepdims=True))
    o_ref[...] = a / nrm


def _stage_c(s2, bg2):
    blk = 5000
    grid = N // blk
    return pl.pallas_call(
        _stage_c_body,
        grid=(grid,),
        in_specs=[
            pl.BlockSpec((blk, D), lambda i: (i, 0)),
            pl.BlockSpec((1, D), lambda i: (0, 0)),
        ],
        out_specs=pl.BlockSpec((blk, D), lambda i: (i, 0)),
        out_shape=jax.ShapeDtypeStruct((N, D), jnp.float32),
    )(s2, bg2.reshape(1, D))


def _sc_body(table_hbm, connt_hbm, out_hbm,
             idx_v, acc0, acc1, acc2, tab_s,
             gsem0, gsem1, gsem2, osem):
    cid = lax.axis_index("c")
    sid = lax.axis_index("s")
    wid = sid * 2 + cid
    accs = (acc0, acc1, acc2, acc0)
    gsems = (gsem0, gsem1, gsem2, gsem0)

    pltpu.sync_copy(connt_hbm.at[:, wid], idx_v)

    @pl.when(sid < 15)
    def _():
        pltpu.sync_copy(table_hbm.at[pl.ds(sid * 624, 624)],
                        tab_s.at[pl.ds(sid * 624, 624)])

    @pl.when(sid == 15)
    def _():
        pltpu.sync_copy(table_hbm.at[pl.ds(9360, 640)],
                        tab_s.at[pl.ds(9360, 640)])

    plsc.subcore_barrier()

    def zero(accb):
        @pl.loop(0, BN)
        def _(r):
            for c in range(LANES):
                accb[r, pl.ds(c * 16, 16)] = jnp.zeros((16,), jnp.float32)

    def issue(b):
        @pl.loop(0, K)
        def _(k):
            pltpu.async_copy(
                tab_s.at[idx_v.at[k, b]],
                accs[b], gsems[b], add=True)

    def drain(b):
        @pl.loop(0, K)
        def _(k):
            pltpu.make_async_copy(
                tab_s.at[idx_v.at[0, 0]],
                accs[b], gsems[b]).wait()

    first = wid * NBLK
    NREAL = N // BN

    for b in range(3):
        @pl.when(first + b < NREAL)
        def _():
            zero(accs[b])
            issue(b)

    for b in range(NBLK):
        @pl.when(first + b < NREAL)
        def _():
            drain(b)
            pltpu.async_copy(
                accs[b],
                out_hbm.at[pl.ds((first + b) * BN, BN)],
                osem).wait()

        if b == 0:
            @pl.when(first + 3 < NREAL)
            def _():
                zero(accs[3])
                issue(3)


@functools.cache
def _sc_gather_sum():
    return pl.kernel(
        _sc_body,
        out_type=jax.ShapeDtypeStruct((N, D), jnp.float32),
        mesh=plsc.VectorSubcoreMesh(core_axis_name="c", subcore_axis_name="s",
                                    num_cores=2, num_subcores=16),
        scratch_types=[
            pltpu.VMEM((K, NBLK, BN), jnp.int32),
            pltpu.VMEM((BN, D), jnp.float32),
            pltpu.VMEM((BN, D), jnp.float32),
            pltpu.VMEM((BN, D), jnp.float32),
            pltpu.VMEM_SHARED((N, D), jnp.float32),
            pltpu.SemaphoreType.DMA,
            pltpu.SemaphoreType.DMA,
            pltpu.SemaphoreType.DMA,
            pltpu.SemaphoreType.DMA,
        ],
    )


def kernel(x, conn, W1, Wg1, bg1, Wg2, bg2):
    conn32 = conn.astype(jnp.int32)

    conn_t = jnp.pad(conn32.T, ((0, 0), (0, N_PAD - N))).reshape(
        K, NW, NBLK, BN)

    sc_gather = _sc_gather_sum()
    g1 = _stage_a(x, W1, Wg1)
    s1 = sc_gather(g1, conn_t)
    g2 = _stage_b(s1, bg1, Wg2)
    s2 = sc_gather(g2, conn_t)
    return _stage_c(s2, bg2)
